# baseline (speedup 1.0000x reference)
import functools

import jax
import jax.numpy as jnp
from jax import lax
from jax.experimental import pallas as pl
from jax.experimental.pallas import tpu as pltpu
from jax.experimental.pallas import tpu_sc as plsc

BATCH = 16384
DIM = 32
NW = 32
BPW = BATCH // NW
CHUNK = 128
NCHUNK = BPW // CHUNK


def _bpr_kernel(user_hbm, pos_hbm, neg_hbm, utab_hbm, itab_hbm,
                pos_out, neg_out,
                uidx_v, pidx_v, nidx_v, urows_v, prows_v, nrows_v,
                pres_v, nres_v, sem):
    nc = 2
    wid = lax.axis_index("s") * nc + lax.axis_index("c")
    base = wid * BPW

    pltpu.sync_copy(user_hbm.at[wid], uidx_v)
    pltpu.sync_copy(pos_hbm.at[wid], pidx_v)
    pltpu.sync_copy(neg_hbm.at[wid], nidx_v)

    copies = []
    for j in range(NCHUNK):
        dst = pl.ds(j * CHUNK, CHUNK)
        copies.append(pltpu.async_copy(utab_hbm.at[uidx_v.at[j]],
                                       urows_v.at[dst], sem))
        copies.append(pltpu.async_copy(itab_hbm.at[pidx_v.at[j]],
                                       prows_v.at[dst], sem))
        copies.append(pltpu.async_copy(itab_hbm.at[nidx_v.at[j]],
                                       nrows_v.at[dst], sem))
    for c in copies:
        c.wait()

    lane = lax.broadcasted_iota(jnp.int32, (16,), 0)
    zero = jnp.zeros((16,), jnp.float32)

    def body(g, _):
        accp = zero
        accn = zero
        for k in range(16):
            i = g * 16 + k
            u0 = urows_v[i, pl.ds(0, 16)]
            u1 = urows_v[i, pl.ds(16, 16)]
            p0 = prows_v[i, pl.ds(0, 16)]
            p1 = prows_v[i, pl.ds(16, 16)]
            n0 = nrows_v[i, pl.ds(0, 16)]
            n1 = nrows_v[i, pl.ds(16, 16)]
            sp = jnp.sum(u0 * p0 + u1 * p1)
            sn = jnp.sum(u0 * n0 + u1 * n1)
            accp = jnp.where(lane == k, sp, accp)
            accn = jnp.where(lane == k, sn, accn)
        pres_v[pl.ds(g * 16, 16)] = accp
        nres_v[pl.ds(g * 16, 16)] = accn
        return _

    lax.fori_loop(0, BPW // 16, body, None)

    pltpu.sync_copy(pres_v, pos_out.at[pl.ds(base, BPW)])
    pltpu.sync_copy(nres_v, neg_out.at[pl.ds(base, BPW)])


@jax.jit
def kernel(user, pos_item, neg_item, user_table, item_table):
    mesh = plsc.VectorSubcoreMesh(core_axis_name="c", subcore_axis_name="s")
    f32 = jnp.float32
    run = functools.partial(
        pl.kernel,
        mesh=mesh,
        compiler_params=pltpu.CompilerParams(
            needs_layout_passes=False, use_tc_tiling_on_sc=False),
        out_type=(jax.ShapeDtypeStruct((BATCH,), f32),
                  jax.ShapeDtypeStruct((BATCH,), f32)),
        scratch_types=[
            pltpu.VMEM((NCHUNK, CHUNK), jnp.int32),
            pltpu.VMEM((NCHUNK, CHUNK), jnp.int32),
            pltpu.VMEM((NCHUNK, CHUNK), jnp.int32),
            pltpu.VMEM((BPW, DIM), f32),
            pltpu.VMEM((BPW, DIM), f32),
            pltpu.VMEM((BPW, DIM), f32),
            pltpu.VMEM((BPW,), f32),
            pltpu.VMEM((BPW,), f32),
            pltpu.SemaphoreType.DMA,
        ],
    )(_bpr_kernel)
    u3 = user.astype(jnp.int32).reshape(NW, NCHUNK, CHUNK)
    p3 = pos_item.astype(jnp.int32).reshape(NW, NCHUNK, CHUNK)
    n3 = neg_item.astype(jnp.int32).reshape(NW, NCHUNK, CHUNK)
    return run(u3, p3, n3, user_table, item_table)


# speedup vs baseline: 1.3580x; 1.3580x over previous
import functools

import jax
import jax.numpy as jnp
from jax import lax
from jax.experimental import pallas as pl
from jax.experimental.pallas import tpu as pltpu
from jax.experimental.pallas import tpu_sc as plsc

NUM_ROWS = 1000000
BATCH = 16384
DIM = 32
NW = 32
BPW = BATCH // NW
NCOLS = 7813
FULLCOLS = 7812
TAIL0 = FULLCOLS * 128
STAGE_ROWS = NCOLS * DIM


def _detile(utab_hbm, itab_hbm, utail_hbm, itail_hbm,
            uout, iout, buf_v, tail_v):
    nc = 2
    wid = lax.axis_index("s") * nc + lax.axis_index("c")

    def run_table(tab, out):
        n = (FULLCOLS - wid + NW - 1) // NW

        def body(k, _):
            c = wid + k * NW
            src = pl.ds(pl.multiple_of(c * 128, 128), 128)
            dst = pl.ds(pl.multiple_of(c * DIM, 8), DIM)
            pltpu.sync_copy(tab.at[:, src], buf_v)
            pltpu.sync_copy(buf_v, out.at[dst, :])
            return _

        lax.fori_loop(0, n, body, None)

    run_table(utab_hbm, uout)
    run_table(itab_hbm, iout)

    tdst = pl.ds(FULLCOLS * DIM, DIM)

    @pl.when(wid == 0)
    def _():
        pltpu.sync_copy(utail_hbm, tail_v)
        pltpu.sync_copy(tail_v, uout.at[tdst, :])

    @pl.when(wid == 1)
    def _():
        pltpu.sync_copy(itail_hbm, tail_v)
        pltpu.sync_copy(tail_v, iout.at[tdst, :])


def _gather_dot(user_hbm, pos_hbm, neg_hbm, utab3, itab3,
                pos_out, neg_out,
                uidx_v, pidx_v, nidx_v, uS, pS, nS,
                pres_v, nres_v, sem):
    nc = 2
    wid = lax.axis_index("s") * nc + lax.axis_index("c")
    base = wid * BPW

    pltpu.sync_copy(user_hbm.at[pl.ds(base, BPW)], uidx_v)
    pltpu.sync_copy(pos_hbm.at[pl.ds(base, BPW)], pidx_v)
    pltpu.sync_copy(neg_hbm.at[pl.ds(base, BPW)], nidx_v)

    lane = lax.broadcasted_iota(jnp.int32, (16,), 0)

    def group(g, _):
        uvec = uidx_v[pl.ds(g * 16, 16)]
        pvec = pidx_v[pl.ds(g * 16, 16)]
        nvec = nidx_v[pl.ds(g * 16, 16)]
        copies = []
        for k in range(16):
            for vec, tab, dstS in ((uvec, utab3, uS), (pvec, itab3, pS),
                                   (nvec, itab3, nS)):
                r = vec[k]
                j = lax.shift_right_logical(r, 7)
                b16 = pl.multiple_of(lax.bitwise_and(r, 112), 16)
                copies.append(
                    pltpu.async_copy(tab.at[j, :, pl.ds(b16, 16)],
                                     dstS.at[k], sem))
        for c in copies:
            c.wait()
        bu = lax.bitwise_and(uvec, 15)
        bp = lax.bitwise_and(pvec, 15)
        bn = lax.bitwise_and(nvec, 15)
        accp = jnp.zeros((16,), jnp.float32)
        accn = jnp.zeros((16,), jnp.float32)
        for d in range(DIM):
            dvec = jnp.full((16,), d, jnp.int32)
            u = plsc.load_gather(uS, [lane, dvec, bu])
            p = plsc.load_gather(pS, [lane, dvec, bp])
            n = plsc.load_gather(nS, [lane, dvec, bn])
            accp = accp + u * p
            accn = accn + u * n
        sl = pl.ds(g * 16, 16)
        pres_v[sl] = accp
        nres_v[sl] = accn
        return _

    lax.fori_loop(0, BPW // 16, group, None)

    pltpu.sync_copy(pres_v, pos_out.at[pl.ds(base, BPW)])
    pltpu.sync_copy(nres_v, neg_out.at[pl.ds(base, BPW)])


@jax.jit
def kernel(user, pos_item, neg_item, user_table, item_table):
    mesh = plsc.VectorSubcoreMesh(core_axis_name="c", subcore_axis_name="s")
    f32 = jnp.float32

    detile = functools.partial(
        pl.kernel,
        mesh=mesh,
        compiler_params=pltpu.CompilerParams(
            needs_layout_passes=False, use_tc_tiling_on_sc=True),
        out_type=(jax.ShapeDtypeStruct((STAGE_ROWS, 128), f32),
                  jax.ShapeDtypeStruct((STAGE_ROWS, 128), f32)),
        scratch_types=[
            pltpu.VMEM((DIM, 128), f32),
            pltpu.VMEM((DIM, 128), f32),
        ],
    )(_detile)

    gather_dot = functools.partial(
        pl.kernel,
        mesh=mesh,
        compiler_params=pltpu.CompilerParams(
            needs_layout_passes=False, use_tc_tiling_on_sc=False),
        out_type=(jax.ShapeDtypeStruct((BATCH,), f32),
                  jax.ShapeDtypeStruct((BATCH,), f32)),
        scratch_types=[
            pltpu.VMEM((BPW,), jnp.int32),
            pltpu.VMEM((BPW,), jnp.int32),
            pltpu.VMEM((BPW,), jnp.int32),
            pltpu.VMEM((16, DIM, 16), f32),
            pltpu.VMEM((16, DIM, 16), f32),
            pltpu.VMEM((16, DIM, 16), f32),
            pltpu.VMEM((BPW,), f32),
            pltpu.VMEM((BPW,), f32),
            pltpu.SemaphoreType.DMA,
        ],
    )(_gather_dot)

    utail = jnp.pad(user_table[TAIL0:], ((0, 64), (0, 0))).T
    itail = jnp.pad(item_table[TAIL0:], ((0, 64), (0, 0))).T
    uflat, iflat = detile(user_table.T, item_table.T, utail, itail)
    utab3 = uflat.reshape(NCOLS, DIM, 128)
    itab3 = iflat.reshape(NCOLS, DIM, 128)
    return gather_dot(user.astype(jnp.int32), pos_item.astype(jnp.int32),
                      neg_item.astype(jnp.int32), utab3, itab3)


# speedup vs baseline: 2.5272x; 1.8610x over previous
import functools

import jax
import jax.numpy as jnp
from jax import lax
from jax.experimental import pallas as pl
from jax.experimental.pallas import tpu as pltpu
from jax.experimental.pallas import tpu_sc as plsc

NUM_ROWS = 1000000
BATCH = 16384
DIM = 32
NW = 32
BPW = BATCH // NW
NCOLS = 7813
FULLCOLS = 7812
TAIL0 = FULLCOLS * 128
STAGE_ROWS = NCOLS * DIM


NBUF = 4
DCHUNKS = 62


def _detile(utab_hbm, itab_hbm, utail_hbm, itail_hbm,
            uout, iout, b0, b1, b2, b3, tail_v,
            r0, r1, r2, r3, w0, w1, w2, w3):
    bufs = (b0, b1, b2, b3)
    rsems = (r0, r1, r2, r3)
    wsems = (w0, w1, w2, w3)
    nc = 2
    wid = lax.axis_index("s") * nc + lax.axis_index("c")

    def run_table(tab, out):
        def chunk(g, _):
            cols = [wid + (g * NBUF + j) * NW for j in range(NBUF)]
            for j in range(NBUF):
                cc = cols[j] - NBUF * NW

                @pl.when(jnp.logical_and(cc >= 0, cc < FULLCOLS))
                def _(j=j):
                    pltpu.make_async_copy(
                        bufs[j], out.at[pl.ds(0, DIM), :], wsems[j]).wait()

            reads = [None] * NBUF
            for j in range(NBUF):
                @pl.when(cols[j] < FULLCOLS)
                def _(j=j):
                    src = pl.ds(pl.multiple_of(cols[j] * 128, 128), 128)
                    pltpu.async_copy(tab.at[:, src], bufs[j], rsems[j])

            for j in range(NBUF):
                @pl.when(cols[j] < FULLCOLS)
                def _(j=j):
                    src = pl.ds(pl.multiple_of(cols[j] * 128, 128), 128)
                    pltpu.make_async_copy(tab.at[:, src], bufs[j],
                                          rsems[j]).wait()
                    dst = pl.ds(pl.multiple_of(cols[j] * DIM, 8), DIM)
                    pltpu.async_copy(bufs[j], out.at[dst, :], wsems[j])
            return _

        lax.fori_loop(0, DCHUNKS, chunk, None)
        for j in range(NBUF):
            cf = wid + ((DCHUNKS - 1) * NBUF + j) * NW

            @pl.when(cf < FULLCOLS)
            def _(j=j):
                pltpu.make_async_copy(
                    bufs[j], out.at[pl.ds(0, DIM), :], wsems[j]).wait()

    run_table(utab_hbm, uout)
    run_table(itab_hbm, iout)

    tdst = pl.ds(FULLCOLS * DIM, DIM)

    @pl.when(wid == 0)
    def _():
        pltpu.sync_copy(utail_hbm, tail_v)
        pltpu.sync_copy(tail_v, uout.at[tdst, :])

    @pl.when(wid == 1)
    def _():
        pltpu.sync_copy(itail_hbm, tail_v)
        pltpu.sync_copy(tail_v, iout.at[tdst, :])


def _gather_dot(user_hbm, pos_hbm, neg_hbm, utab3, itab3,
                pos_out, neg_out,
                uidx_v, pidx_v, nidx_v, uS, pS, nS,
                pres_v, nres_v, sem):
    nc = 2
    wid = lax.axis_index("s") * nc + lax.axis_index("c")
    base = wid * BPW

    pltpu.sync_copy(user_hbm.at[pl.ds(base, BPW)], uidx_v)
    pltpu.sync_copy(pos_hbm.at[pl.ds(base, BPW)], pidx_v)
    pltpu.sync_copy(neg_hbm.at[pl.ds(base, BPW)], nidx_v)

    lane = lax.broadcasted_iota(jnp.int32, (16,), 0)

    def group(g, _):
        uvec = uidx_v[pl.ds(g * 16, 16)]
        pvec = pidx_v[pl.ds(g * 16, 16)]
        nvec = nidx_v[pl.ds(g * 16, 16)]
        copies = []
        for k in range(16):
            for vec, tab, dstS in ((uvec, utab3, uS), (pvec, itab3, pS),
                                   (nvec, itab3, nS)):
                r = vec[k]
                j = lax.shift_right_logical(r, 7)
                b16 = pl.multiple_of(lax.bitwise_and(r, 112), 16)
                copies.append(
                    pltpu.async_copy(tab.at[j, :, pl.ds(b16, 16)],
                                     dstS.at[k], sem))
        for c in copies:
            c.wait()
        bu = lax.bitwise_and(uvec, 15)
        bp = lax.bitwise_and(pvec, 15)
        bn = lax.bitwise_and(nvec, 15)
        accp = jnp.zeros((16,), jnp.float32)
        accn = jnp.zeros((16,), jnp.float32)
        for d in range(DIM):
            dvec = jnp.full((16,), d, jnp.int32)
            u = plsc.load_gather(uS, [lane, dvec, bu])
            p = plsc.load_gather(pS, [lane, dvec, bp])
            n = plsc.load_gather(nS, [lane, dvec, bn])
            accp = accp + u * p
            accn = accn + u * n
        sl = pl.ds(g * 16, 16)
        pres_v[sl] = accp
        nres_v[sl] = accn
        return _

    lax.fori_loop(0, BPW // 16, group, None)

    pltpu.sync_copy(pres_v, pos_out.at[pl.ds(base, BPW)])
    pltpu.sync_copy(nres_v, neg_out.at[pl.ds(base, BPW)])


@jax.jit
def kernel(user, pos_item, neg_item, user_table, item_table):
    mesh = plsc.VectorSubcoreMesh(core_axis_name="c", subcore_axis_name="s")
    f32 = jnp.float32

    detile = functools.partial(
        pl.kernel,
        mesh=mesh,
        compiler_params=pltpu.CompilerParams(
            needs_layout_passes=False, use_tc_tiling_on_sc=True),
        out_type=(jax.ShapeDtypeStruct((STAGE_ROWS, 128), f32),
                  jax.ShapeDtypeStruct((STAGE_ROWS, 128), f32)),
        scratch_types=(
            [pltpu.VMEM((DIM, 128), f32)] * (NBUF + 1)
            + [pltpu.SemaphoreType.DMA] * (2 * NBUF)
        ),
    )(_detile)

    gather_dot = functools.partial(
        pl.kernel,
        mesh=mesh,
        compiler_params=pltpu.CompilerParams(
            needs_layout_passes=False, use_tc_tiling_on_sc=False),
        out_type=(jax.ShapeDtypeStruct((BATCH,), f32),
                  jax.ShapeDtypeStruct((BATCH,), f32)),
        scratch_types=[
            pltpu.VMEM((BPW,), jnp.int32),
            pltpu.VMEM((BPW,), jnp.int32),
            pltpu.VMEM((BPW,), jnp.int32),
            pltpu.VMEM((16, DIM, 16), f32),
            pltpu.VMEM((16, DIM, 16), f32),
            pltpu.VMEM((16, DIM, 16), f32),
            pltpu.VMEM((BPW,), f32),
            pltpu.VMEM((BPW,), f32),
            pltpu.SemaphoreType.DMA,
        ],
    )(_gather_dot)

    utail = jnp.pad(user_table[TAIL0:], ((0, 64), (0, 0))).T
    itail = jnp.pad(item_table[TAIL0:], ((0, 64), (0, 0))).T
    uflat, iflat = detile(user_table.T, item_table.T, utail, itail)
    utab3 = uflat.reshape(NCOLS, DIM, 128)
    itab3 = iflat.reshape(NCOLS, DIM, 128)
    return gather_dot(user.astype(jnp.int32), pos_item.astype(jnp.int32),
                      neg_item.astype(jnp.int32), utab3, itab3)


# speedup vs baseline: 3.0006x; 1.1873x over previous
import functools

import jax
import jax.numpy as jnp
from jax import lax
from jax.experimental import pallas as pl
from jax.experimental.pallas import tpu as pltpu
from jax.experimental.pallas import tpu_sc as plsc

NUM_ROWS = 1000000
BATCH = 16384
DIM = 32
NW = 32
BPW = BATCH // NW
NCOLS = 7813
FULLCOLS = 7812
TAIL0 = FULLCOLS * 128
STAGE_ROWS = NCOLS * DIM


NBUF = 8
DCHUNKS = 31


def _detile(utab_hbm, itab_hbm, utail_hbm, itail_hbm,
            uout, iout, b0, b1, b2, b3, b4, b5, b6, b7, tail_v,
            r0, r1, r2, r3, r4, r5, r6, r7,
            w0, w1, w2, w3, w4, w5, w6, w7):
    bufs = (b0, b1, b2, b3, b4, b5, b6, b7)
    rsems = (r0, r1, r2, r3, r4, r5, r6, r7)
    wsems = (w0, w1, w2, w3, w4, w5, w6, w7)
    nc = 2
    wid = lax.axis_index("s") * nc + lax.axis_index("c")

    def run_table(tab, out):
        def chunk(g, _):
            cols = [wid + (g * NBUF + j) * NW for j in range(NBUF)]
            for j in range(NBUF):
                cc = cols[j] - NBUF * NW

                @pl.when(jnp.logical_and(cc >= 0, cc < FULLCOLS))
                def _(j=j):
                    pltpu.make_async_copy(
                        bufs[j], out.at[pl.ds(0, DIM), :], wsems[j]).wait()

            reads = [None] * NBUF
            for j in range(NBUF):
                @pl.when(cols[j] < FULLCOLS)
                def _(j=j):
                    src = pl.ds(pl.multiple_of(cols[j] * 128, 128), 128)
                    pltpu.async_copy(tab.at[:, src], bufs[j], rsems[j])

            for j in range(NBUF):
                @pl.when(cols[j] < FULLCOLS)
                def _(j=j):
                    src = pl.ds(pl.multiple_of(cols[j] * 128, 128), 128)
                    pltpu.make_async_copy(tab.at[:, src], bufs[j],
                                          rsems[j]).wait()
                    dst = pl.ds(pl.multiple_of(cols[j] * DIM, 8), DIM)
                    pltpu.async_copy(bufs[j], out.at[dst, :], wsems[j])
            return _

        lax.fori_loop(0, DCHUNKS, chunk, None)
        for j in range(NBUF):
            cf = wid + ((DCHUNKS - 1) * NBUF + j) * NW

            @pl.when(cf < FULLCOLS)
            def _(j=j):
                pltpu.make_async_copy(
                    bufs[j], out.at[pl.ds(0, DIM), :], wsems[j]).wait()

    run_table(utab_hbm, uout)
    run_table(itab_hbm, iout)

    tdst = pl.ds(FULLCOLS * DIM, DIM)

    @pl.when(wid == 0)
    def _():
        pltpu.sync_copy(utail_hbm, tail_v)
        pltpu.sync_copy(tail_v, uout.at[tdst, :])

    @pl.when(wid == 1)
    def _():
        pltpu.sync_copy(itail_hbm, tail_v)
        pltpu.sync_copy(tail_v, iout.at[tdst, :])


def _gather_dot(user_hbm, pos_hbm, neg_hbm, utab3, itab3,
                pos_out, neg_out,
                uidx_v, pidx_v, nidx_v, uS, pS, nS,
                pres_v, nres_v, sem):
    nc = 2
    wid = lax.axis_index("s") * nc + lax.axis_index("c")
    base = wid * BPW

    pltpu.sync_copy(user_hbm.at[pl.ds(base, BPW)], uidx_v)
    pltpu.sync_copy(pos_hbm.at[pl.ds(base, BPW)], pidx_v)
    pltpu.sync_copy(neg_hbm.at[pl.ds(base, BPW)], nidx_v)

    lane = lax.broadcasted_iota(jnp.int32, (16,), 0)

    def group(g, _):
        uvec = uidx_v[pl.ds(g * 16, 16)]
        pvec = pidx_v[pl.ds(g * 16, 16)]
        nvec = nidx_v[pl.ds(g * 16, 16)]
        copies = []
        for k in range(16):
            for vec, tab, dstS in ((uvec, utab3, uS), (pvec, itab3, pS),
                                   (nvec, itab3, nS)):
                r = vec[k]
                j = lax.shift_right_logical(r, 7)
                b16 = pl.multiple_of(lax.bitwise_and(r, 112), 16)
                copies.append(
                    pltpu.async_copy(tab.at[j, :, pl.ds(b16, 16)],
                                     dstS.at[k], sem))
        for c in copies:
            c.wait()
        bu = lax.bitwise_and(uvec, 15)
        bp = lax.bitwise_and(pvec, 15)
        bn = lax.bitwise_and(nvec, 15)
        accp = jnp.zeros((16,), jnp.float32)
        accn = jnp.zeros((16,), jnp.float32)
        for d in range(DIM):
            dvec = jnp.full((16,), d, jnp.int32)
            u = plsc.load_gather(uS, [lane, dvec, bu])
            p = plsc.load_gather(pS, [lane, dvec, bp])
            n = plsc.load_gather(nS, [lane, dvec, bn])
            accp = accp + u * p
            accn = accn + u * n
        sl = pl.ds(g * 16, 16)
        pres_v[sl] = accp
        nres_v[sl] = accn
        return _

    lax.fori_loop(0, BPW // 16, group, None)

    pltpu.sync_copy(pres_v, pos_out.at[pl.ds(base, BPW)])
    pltpu.sync_copy(nres_v, neg_out.at[pl.ds(base, BPW)])


@jax.jit
def kernel(user, pos_item, neg_item, user_table, item_table):
    mesh = plsc.VectorSubcoreMesh(core_axis_name="c", subcore_axis_name="s")
    f32 = jnp.float32

    detile = functools.partial(
        pl.kernel,
        mesh=mesh,
        compiler_params=pltpu.CompilerParams(
            needs_layout_passes=False, use_tc_tiling_on_sc=True),
        out_type=(jax.ShapeDtypeStruct((STAGE_ROWS, 128), f32),
                  jax.ShapeDtypeStruct((STAGE_ROWS, 128), f32)),
        scratch_types=(
            [pltpu.VMEM((DIM, 128), f32)] * (NBUF + 1)
            + [pltpu.SemaphoreType.DMA] * (2 * NBUF)
        ),
    )(_detile)

    gather_dot = functools.partial(
        pl.kernel,
        mesh=mesh,
        compiler_params=pltpu.CompilerParams(
            needs_layout_passes=False, use_tc_tiling_on_sc=False),
        out_type=(jax.ShapeDtypeStruct((BATCH,), f32),
                  jax.ShapeDtypeStruct((BATCH,), f32)),
        scratch_types=[
            pltpu.VMEM((BPW,), jnp.int32),
            pltpu.VMEM((BPW,), jnp.int32),
            pltpu.VMEM((BPW,), jnp.int32),
            pltpu.VMEM((16, DIM, 16), f32),
            pltpu.VMEM((16, DIM, 16), f32),
            pltpu.VMEM((16, DIM, 16), f32),
            pltpu.VMEM((BPW,), f32),
            pltpu.VMEM((BPW,), f32),
            pltpu.SemaphoreType.DMA,
        ],
    )(_gather_dot)

    utail = jnp.pad(user_table[TAIL0:], ((0, 64), (0, 0))).T
    itail = jnp.pad(item_table[TAIL0:], ((0, 64), (0, 0))).T
    uflat, iflat = detile(user_table.T, item_table.T, utail, itail)
    utab3 = uflat.reshape(NCOLS, DIM, 128)
    itab3 = iflat.reshape(NCOLS, DIM, 128)
    return gather_dot(user.astype(jnp.int32), pos_item.astype(jnp.int32),
                      neg_item.astype(jnp.int32), utab3, itab3)


# speedup vs baseline: 3.2394x; 1.0796x over previous
import functools

import jax
import jax.numpy as jnp
from jax import lax
from jax.experimental import pallas as pl
from jax.experimental.pallas import tpu as pltpu
from jax.experimental.pallas import tpu_sc as plsc

NUM_ROWS = 1000000
BATCH = 16384
DIM = 32
NW = 32
BPW = BATCH // NW
NCOLS = 7813
FULLCOLS = 7812
TAIL0 = FULLCOLS * 128
STAGE_ROWS = NCOLS * DIM


NBUF = 8
DCHUNKS = 31


def _detile(utab_hbm, itab_hbm, utail_hbm, itail_hbm,
            uout, iout, b0, b1, b2, b3, b4, b5, b6, b7, tail_v,
            r0, r1, r2, r3, r4, r5, r6, r7,
            w0, w1, w2, w3, w4, w5, w6, w7):
    bufs = (b0, b1, b2, b3, b4, b5, b6, b7)
    rsems = (r0, r1, r2, r3, r4, r5, r6, r7)
    wsems = (w0, w1, w2, w3, w4, w5, w6, w7)
    nc = 2
    wid = lax.axis_index("s") * nc + lax.axis_index("c")

    def run_table(tab, out):
        def chunk(g, _):
            cols = [wid + (g * NBUF + j) * NW for j in range(NBUF)]
            for j in range(NBUF):
                cc = cols[j] - NBUF * NW

                @pl.when(jnp.logical_and(cc >= 0, cc < FULLCOLS))
                def _(j=j):
                    pltpu.make_async_copy(
                        bufs[j], out.at[pl.ds(0, DIM), :], wsems[j]).wait()

            reads = [None] * NBUF
            for j in range(NBUF):
                @pl.when(cols[j] < FULLCOLS)
                def _(j=j):
                    src = pl.ds(pl.multiple_of(cols[j] * 128, 128), 128)
                    pltpu.async_copy(tab.at[:, src], bufs[j], rsems[j])

            for j in range(NBUF):
                @pl.when(cols[j] < FULLCOLS)
                def _(j=j):
                    src = pl.ds(pl.multiple_of(cols[j] * 128, 128), 128)
                    pltpu.make_async_copy(tab.at[:, src], bufs[j],
                                          rsems[j]).wait()
                    dst = pl.ds(pl.multiple_of(cols[j] * DIM, 8), DIM)
                    pltpu.async_copy(bufs[j], out.at[dst, :], wsems[j])
            return _

        lax.fori_loop(0, DCHUNKS, chunk, None)
        for j in range(NBUF):
            cf = wid + ((DCHUNKS - 1) * NBUF + j) * NW

            @pl.when(cf < FULLCOLS)
            def _(j=j):
                pltpu.make_async_copy(
                    bufs[j], out.at[pl.ds(0, DIM), :], wsems[j]).wait()

    run_table(utab_hbm, uout)
    run_table(itab_hbm, iout)

    tdst = pl.ds(FULLCOLS * DIM, DIM)

    @pl.when(wid == 0)
    def _():
        pltpu.sync_copy(utail_hbm, tail_v)
        pltpu.sync_copy(tail_v, uout.at[tdst, :])

    @pl.when(wid == 1)
    def _():
        pltpu.sync_copy(itail_hbm, tail_v)
        pltpu.sync_copy(tail_v, iout.at[tdst, :])


def _gather_dot(user_hbm, pos_hbm, neg_hbm, utab3, itab3,
                pos_out, neg_out,
                uidx_v, pidx_v, nidx_v,
                uSA, pSA, nSA, uSB, pSB, nSB,
                pres_v, nres_v, semA, semB):
    nc = 2
    wid = lax.axis_index("s") * nc + lax.axis_index("c")
    base = wid * BPW

    pltpu.sync_copy(user_hbm.at[pl.ds(base, BPW)], uidx_v)
    pltpu.sync_copy(pos_hbm.at[pl.ds(base, BPW)], pidx_v)
    pltpu.sync_copy(neg_hbm.at[pl.ds(base, BPW)], nidx_v)

    lane = lax.broadcasted_iota(jnp.int32, (16,), 0)
    setA = (uSA, pSA, nSA, semA)
    setB = (uSB, pSB, nSB, semB)
    dummy = utab3.at[pl.ds(0, 16), :, pl.ds(0, 16)]

    def fire(g, bufset):
        uS, pS, nS, sem = bufset
        uvec = uidx_v[pl.ds(g * 16, 16)]
        pvec = pidx_v[pl.ds(g * 16, 16)]
        nvec = nidx_v[pl.ds(g * 16, 16)]
        for k in range(16):
            for vec, tab, dstS in ((uvec, utab3, uS), (pvec, itab3, pS),
                                   (nvec, itab3, nS)):
                r = vec[k]
                j = lax.shift_right_logical(r, 7)
                b16 = pl.multiple_of(lax.bitwise_and(r, 112), 16)
                pltpu.async_copy(tab.at[j, :, pl.ds(b16, 16)],
                                 dstS.at[k], sem)

    def compute(g, bufset):
        uS, pS, nS, sem = bufset
        for dstS in (uS, pS, nS):
            pltpu.make_async_copy(dummy, dstS, sem).wait()
        uvec = uidx_v[pl.ds(g * 16, 16)]
        pvec = pidx_v[pl.ds(g * 16, 16)]
        nvec = nidx_v[pl.ds(g * 16, 16)]
        bu = lax.bitwise_and(uvec, 15)
        bp = lax.bitwise_and(pvec, 15)
        bn = lax.bitwise_and(nvec, 15)
        accp = jnp.zeros((16,), jnp.float32)
        accn = jnp.zeros((16,), jnp.float32)
        for d in range(DIM):
            dvec = jnp.full((16,), d, jnp.int32)
            u = plsc.load_gather(uS, [lane, dvec, bu])
            p = plsc.load_gather(pS, [lane, dvec, bp])
            n = plsc.load_gather(nS, [lane, dvec, bn])
            accp = accp + u * p
            accn = accn + u * n
        sl = pl.ds(g * 16, 16)
        pres_v[sl] = accp
        nres_v[sl] = accn

    NG = BPW // 16
    fire(0, setA)

    def pair(h, _):
        g0 = h * 2
        fire(g0 + 1, setB)
        compute(g0, setA)

        @pl.when(g0 + 2 < NG)
        def _():
            fire(g0 + 2, setA)

        compute(g0 + 1, setB)
        return _

    lax.fori_loop(0, NG // 2, pair, None)

    pltpu.sync_copy(pres_v, pos_out.at[pl.ds(base, BPW)])
    pltpu.sync_copy(nres_v, neg_out.at[pl.ds(base, BPW)])


@jax.jit
def kernel(user, pos_item, neg_item, user_table, item_table):
    mesh = plsc.VectorSubcoreMesh(core_axis_name="c", subcore_axis_name="s")
    f32 = jnp.float32

    detile = functools.partial(
        pl.kernel,
        mesh=mesh,
        compiler_params=pltpu.CompilerParams(
            needs_layout_passes=False, use_tc_tiling_on_sc=True),
        out_type=(jax.ShapeDtypeStruct((STAGE_ROWS, 128), f32),
                  jax.ShapeDtypeStruct((STAGE_ROWS, 128), f32)),
        scratch_types=(
            [pltpu.VMEM((DIM, 128), f32)] * (NBUF + 1)
            + [pltpu.SemaphoreType.DMA] * (2 * NBUF)
        ),
    )(_detile)

    gather_dot = functools.partial(
        pl.kernel,
        mesh=mesh,
        compiler_params=pltpu.CompilerParams(
            needs_layout_passes=False, use_tc_tiling_on_sc=False),
        out_type=(jax.ShapeDtypeStruct((BATCH,), f32),
                  jax.ShapeDtypeStruct((BATCH,), f32)),
        scratch_types=[
            pltpu.VMEM((BPW,), jnp.int32),
            pltpu.VMEM((BPW,), jnp.int32),
            pltpu.VMEM((BPW,), jnp.int32),
            pltpu.VMEM((16, DIM, 16), f32),
            pltpu.VMEM((16, DIM, 16), f32),
            pltpu.VMEM((16, DIM, 16), f32),
            pltpu.VMEM((16, DIM, 16), f32),
            pltpu.VMEM((16, DIM, 16), f32),
            pltpu.VMEM((16, DIM, 16), f32),
            pltpu.VMEM((BPW,), f32),
            pltpu.VMEM((BPW,), f32),
            pltpu.SemaphoreType.DMA,
            pltpu.SemaphoreType.DMA,
        ],
    )(_gather_dot)

    utail = jnp.pad(user_table[TAIL0:], ((0, 64), (0, 0))).T
    itail = jnp.pad(item_table[TAIL0:], ((0, 64), (0, 0))).T
    uflat, iflat = detile(user_table.T, item_table.T, utail, itail)
    utab3 = uflat.reshape(NCOLS, DIM, 128)
    itab3 = iflat.reshape(NCOLS, DIM, 128)
    return gather_dot(user.astype(jnp.int32), pos_item.astype(jnp.int32),
                      neg_item.astype(jnp.int32), utab3, itab3)
